# Initial kernel scaffold; baseline (speedup 1.0000x reference)
import jax
import jax.numpy as jnp
from jax.experimental import pallas as pl


def kernel(x, edge_index, batch, W1, b1, W2, b2, W3, b3, Wl, bl):
    raise NotImplementedError("write your pallas kernel here")


# speedup vs baseline: 7.0807x; 7.0807x over previous
import functools

import jax
import jax.numpy as jnp
from jax import lax
from jax.experimental import pallas as pl
from jax.experimental.pallas import tpu as pltpu
from jax.experimental.pallas import tpu_sc as plsc

_NR = 10000
_E = 320000
_EP = 327680
_CH_E = 160
_CH_DEG = 80
_CB = 40
_ACC = 10112
_XO = 10400
_BN = 400
_NB = _NR // _BN
_NBA = _XO // _BN
_G = 128
_GACC = 256
_EP_P = 10240
_CH_P = 5
_H = 256
_D = 128
_C = 10


def _sc_agg(nacc, w2, ch, xoff):
    mesh = plsc.VectorSubcoreMesh(core_axis_name="c", subcore_axis_name="s")
    nt = nacc // 16

    @functools.partial(
        pl.kernel,
        out_type=jax.ShapeDtypeStruct((2 * xoff, w2), jnp.float32),
        mesh=mesh,
        scratch_types=[
            pltpu.VMEM((min(ch, _CB), 128), jnp.int32),
            pltpu.VMEM((min(ch, _CB), 128), jnp.int32),
            pltpu.VMEM((128, w2), jnp.float32),
            pltpu.VMEM_SHARED((nacc, w2), jnp.float32),
            pltpu.SemaphoreType.DMA,
        ],
    )
    def k(src_h, dst_h, tab_h, zer_h, out_h, sidx, didx, rows, acc, sem):
        c = lax.axis_index("c")
        s = lax.axis_index("s")
        cb = min(ch, _CB)
        pltpu.sync_copy(zer_h, acc.at[pl.ds(s * nt, nt)])
        plsc.subcore_barrier()

        def blk(b, carry):
            pltpu.sync_copy(dst_h.at[c, s, pl.ds(b * cb, cb)], didx)
            pltpu.sync_copy(src_h.at[c, s, pl.ds(b * cb, cb)], sidx)

            def step(j, carry2):
                pltpu.async_copy(tab_h.at[sidx.at[j]], rows, sem).wait()
                pltpu.sync_copy(rows, acc.at[didx.at[j]], add=True)
                return carry2

            return lax.fori_loop(0, cb, step, carry)

        lax.fori_loop(0, ch // cb, blk, 0)

        plsc.subcore_barrier()
        pltpu.sync_copy(acc.at[pl.ds(s * nt, nt)],
                        out_h.at[pl.ds(c * xoff + s * nt, nt)])

    return k


def _sc_count(nacc, w2, ch, xoff):
    mesh = plsc.VectorSubcoreMesh(core_axis_name="c", subcore_axis_name="s")
    nt = nacc // 16

    @functools.partial(
        pl.kernel,
        out_type=jax.ShapeDtypeStruct((2 * xoff, w2), jnp.float32),
        mesh=mesh,
        scratch_types=[
            pltpu.VMEM((ch, 128), jnp.int32),
            pltpu.VMEM((128, w2), jnp.float32),
            pltpu.VMEM_SHARED((nacc, w2), jnp.float32),
        ],
    )
    def k(dst_h, one_h, zer_h, out_h, didx, rows, acc):
        c = lax.axis_index("c")
        s = lax.axis_index("s")
        pltpu.sync_copy(zer_h, acc.at[pl.ds(s * nt, nt)])
        pltpu.sync_copy(dst_h.at[c, s], didx)
        pltpu.sync_copy(one_h, rows)
        plsc.subcore_barrier()

        def step(j, carry):
            pltpu.sync_copy(rows, acc.at[didx.at[j]], add=True)
            return carry

        lax.fori_loop(0, ch, step, 0)

        plsc.subcore_barrier()
        pltpu.sync_copy(acc.at[pl.ds(s * nt, nt)],
                        out_h.at[pl.ds(c * xoff + s * nt, nt)])

    return k


def _tck_entry(x, deg2):
    def body(x_ref, d0_ref, d1_ref, y_ref, dis_ref):
        d = lax.rsqrt(d0_ref[:, 0:1] + d1_ref[:, 0:1] + 1.0)
        y_ref[...] = x_ref[...] * d
        dis_ref[...] = jnp.broadcast_to(d, (_BN, 16))

    return pl.pallas_call(
        body,
        grid=(_NB,),
        in_specs=[
            pl.BlockSpec((_BN, 128), lambda i: (i, 0)),
            pl.BlockSpec((_BN, 128), lambda i: (i, 0)),
            pl.BlockSpec((_BN, 128), lambda i: (_NBA + i, 0)),
        ],
        out_specs=[
            pl.BlockSpec((_BN, 128), lambda i: (i, 0)),
            pl.BlockSpec((_BN, 16), lambda i: (i, 0)),
        ],
        out_shape=[
            jax.ShapeDtypeStruct((_NR, 128), jnp.float32),
            jax.ShapeDtypeStruct((_NR, 16), jnp.float32),
        ],
    )(x, deg2, deg2)


def _tck_layer1(a_esplit, y1, dis16, W, b):
    def body(a0, a1, y, dis, w_ref, b_ref, o_ref):
        d = dis[:, 0:1]
        z = (a0[...] + a1[...] + y[...]) * d
        h = jnp.dot(z, w_ref[...], preferred_element_type=jnp.float32)
        h = jnp.maximum(h + b_ref[0:1, :], 0.0) * d
        o_ref[...] = h

    return pl.pallas_call(
        body,
        grid=(_NB, 2),
        in_specs=[
            pl.BlockSpec((_BN, 128), lambda i, c: (i, 0)),
            pl.BlockSpec((_BN, 128), lambda i, c: (_NBA + i, 0)),
            pl.BlockSpec((_BN, 128), lambda i, c: (i, 0)),
            pl.BlockSpec((_BN, 16), lambda i, c: (i, 0)),
            pl.BlockSpec((_D, 128), lambda i, c: (0, c)),
            pl.BlockSpec((8, 128), lambda i, c: (0, c)),
        ],
        out_specs=pl.BlockSpec((_BN, 128), lambda i, c: (c * _NB + i, 0)),
        out_shape=jax.ShapeDtypeStruct((2 * _NR, 128), jnp.float32),
    )(a_esplit, a_esplit, y1, dis16, W, b)


def _tck_layer(a_pair, y_pair, dis16, W, b, relu, post, w2i):
    nya = (2 * _NR) // _BN // 2

    def body(a0, a1, y0, y1, dis, w_ref, b_ref, o_ref):
        d = dis[:, 0:1]
        z = jnp.concatenate([a0[...] + y0[...], a1[...] + y1[...]], axis=1) * d
        h = jnp.dot(z, w_ref[...], preferred_element_type=jnp.float32)
        h = h + b_ref[0:1, :]
        if relu:
            h = jnp.maximum(h, 0.0)
        if post:
            h = h * d
        o_ref[...] = h

    wi = 2 * w2i
    return pl.pallas_call(
        body,
        grid=(_NB, 2),
        in_specs=[
            pl.BlockSpec((_BN, w2i), lambda i, c: (i, 0)),
            pl.BlockSpec((_BN, w2i), lambda i, c: (_NBA + i, 0)),
            pl.BlockSpec((_BN, w2i), lambda i, c: (i, 0)),
            pl.BlockSpec((_BN, w2i), lambda i, c: (nya + i, 0)),
            pl.BlockSpec((_BN, 16), lambda i, c: (i, 0)),
            pl.BlockSpec((wi, 128), lambda i, c: (0, c)),
            pl.BlockSpec((8, 128), lambda i, c: (0, c)),
        ],
        out_specs=pl.BlockSpec((_BN, 128), lambda i, c: (c * _NB + i, 0)),
        out_shape=jax.ShapeDtypeStruct((2 * _NR, 128), jnp.float32),
    )(a_pair, a_pair, y_pair, y_pair, dis16, W, b)


def _tck_head(s_pair, cnt2, Wl_pad, bl_pad):
    def body(s0, s1, cnt, w_ref, b_ref, o_ref):
        c = jnp.maximum(cnt[:, 0:1], 1.0)
        s = jnp.concatenate([s0[...], s1[...]], axis=1) / c
        o_ref[...] = (jnp.dot(s, w_ref[...], preferred_element_type=jnp.float32)
                      + b_ref[0:1, :])

    return pl.pallas_call(
        body,
        grid=(1,),
        in_specs=[
            pl.BlockSpec((_G, 128), lambda i: (0, 0)),
            pl.BlockSpec((_G, 128), lambda i: (2, 0)),
            pl.BlockSpec((_G, 128), lambda i: (0, 0)),
            pl.BlockSpec((_H, 128), lambda i: (0, 0)),
            pl.BlockSpec((8, 128), lambda i: (0, 0)),
        ],
        out_specs=pl.BlockSpec((_G, 128), lambda i: (0, 0)),
        out_shape=jax.ShapeDtypeStruct((_G, 128), jnp.float32),
    )(s_pair, s_pair, cnt2, Wl_pad, bl_pad)


def kernel(x, edge_index, batch, W1, b1, W2, b2, W3, b3, Wl, bl):
    f32 = jnp.float32
    src = edge_index[0].astype(jnp.int32)
    dst = edge_index[1].astype(jnp.int32)
    pad = _EP - _E
    srcp = jnp.concatenate([src, jnp.zeros((pad,), jnp.int32)])
    dstp = jnp.concatenate([dst, jnp.full((pad,), _NR, jnp.int32)])
    src2 = jnp.stack([srcp, srcp + _NR]).reshape(2, 16, _CH_E, 128)
    dst_dup = jnp.broadcast_to(
        dstp.reshape(1, 16, _CH_E, 128), (2, 16, _CH_E, 128))
    src_split = srcp.reshape(2, 16, _CH_DEG, 128)
    dst_split = dstp.reshape(2, 16, _CH_DEG, 128)

    nodes = jnp.arange(_NR, dtype=jnp.int32)
    padp = _EP_P - _NR
    nodesp = jnp.concatenate([nodes, jnp.zeros((padp,), jnp.int32)])
    src2_p = jnp.stack([nodesp, nodesp + _NR]).reshape(2, 16, _CH_P, 128)
    batchp = jnp.concatenate(
        [batch.astype(jnp.int32), jnp.full((padp,), _G, jnp.int32)])
    dst_p = jnp.broadcast_to(
        batchp.reshape(1, 16, _CH_P, 128), (2, 16, _CH_P, 128))

    ones128 = jnp.ones((128, 128), f32)
    z_a = jnp.zeros((_ACC // 16, 128), f32)
    z_p = jnp.zeros((_GACC // 16, 128), f32)

    b1p = jnp.broadcast_to(b1.reshape(1, _H), (8, _H))
    b2p = jnp.broadcast_to(b2.reshape(1, _H), (8, _H))
    b3p = jnp.broadcast_to(b3.reshape(1, _H), (8, _H))
    Wl_pad = jnp.zeros((_H, 128), f32).at[:, :_C].set(Wl)
    bl_pad = jnp.broadcast_to(
        jnp.zeros((128,), f32).at[:_C].set(bl).reshape(1, 128), (8, 128))

    deg2 = _sc_count(_ACC, 128, _CH_DEG, _XO)(dst_split, ones128, z_a)
    cnt2 = _sc_count(_GACC, 128, _CH_P, _GACC)(dst_p, ones128, z_p)

    y1, dis16 = _tck_entry(x, deg2)
    a1 = _sc_agg(_ACC, 128, _CH_DEG, _XO)(src_split, dst_split, y1, z_a)
    y2 = _tck_layer1(a1, y1, dis16, W1, b1p)
    a2 = _sc_agg(_ACC, 128, _CH_E, _XO)(src2, dst_dup, y2, z_a)
    y3 = _tck_layer(a2, y2, dis16, W2, b2p, relu=True, post=True, w2i=128)
    a3 = _sc_agg(_ACC, 128, _CH_E, _XO)(src2, dst_dup, y3, z_a)
    h3 = _tck_layer(a3, y3, dis16, W3, b3p, relu=False, post=False, w2i=128)
    s2 = _sc_agg(_GACC, 128, _CH_P, _GACC)(src2_p, dst_p, h3, z_p)
    out = _tck_head(s2, cnt2, Wl_pad, bl_pad)
    return out[:, :_C]


# speedup vs baseline: 7.4995x; 1.0591x over previous
import functools

import jax
import jax.numpy as jnp
from jax import lax
from jax.experimental import pallas as pl
from jax.experimental.pallas import tpu as pltpu
from jax.experimental.pallas import tpu_sc as plsc

_NR = 10000
_E = 320000
_EP = 327680
_CH_E = 160
_CH_DEG = 80
_CB = 40
_ACC = 10112
_XO = 10400
_BN = 400
_NB = _NR // _BN
_NBA = _XO // _BN
_G = 128
_GACC = 256
_EP_P = 12288
_CH_P = 6
_H = 256
_D = 128
_C = 10


def _sc_agg(nacc, w2, ch, xoff):
    mesh = plsc.VectorSubcoreMesh(core_axis_name="c", subcore_axis_name="s")
    nt = nacc // 16

    @functools.partial(
        pl.kernel,
        out_type=jax.ShapeDtypeStruct((2 * xoff, w2), jnp.float32),
        mesh=mesh,
        scratch_types=[
            pltpu.VMEM((min(ch, _CB), 128), jnp.int32),
            pltpu.VMEM((min(ch, _CB), 128), jnp.int32),
            pltpu.VMEM((128, w2), jnp.float32),
            pltpu.VMEM((128, w2), jnp.float32),
            pltpu.VMEM_SHARED((nacc, w2), jnp.float32),
            pltpu.SemaphoreType.DMA,
            pltpu.SemaphoreType.DMA,
            pltpu.SemaphoreType.DMA,
            pltpu.SemaphoreType.DMA,
        ],
    )
    def k(src_h, dst_h, tab_h, zer_h, out_h, sidx, didx, rows0, rows1,
          acc, g0, g1, s0, s1):
        c = lax.axis_index("c")
        s = lax.axis_index("s")
        cb = min(ch, _CB)
        pltpu.sync_copy(zer_h, acc.at[pl.ds(s * nt, nt)])
        plsc.subcore_barrier()

        def gat(j, rows, sem):
            pltpu.async_copy(tab_h.at[sidx.at[j]], rows, sem)

        def gat_w(j, rows, sem):
            pltpu.make_async_copy(tab_h.at[sidx.at[j]], rows, sem).wait()

        def sca(j, rows, sem):
            pltpu.async_copy(rows, acc.at[didx.at[j]], sem, add=True)

        def sca_w(j, rows, sem):
            pltpu.make_async_copy(rows, acc.at[didx.at[j]], sem).wait()

        def blk(b, carry):
            pltpu.sync_copy(dst_h.at[c, s, pl.ds(b * cb, cb)], didx)
            pltpu.sync_copy(src_h.at[c, s, pl.ds(b * cb, cb)], sidx)
            gat(0, rows0, g0)

            def pair(t, carry2):
                j0 = 2 * t
                j1 = j0 + 1
                gat_w(j0, rows0, g0)
                gat(j1, rows1, g1)
                sca(j0, rows0, s0)
                gat_w(j1, rows1, g1)
                sca_w(j0, rows0, s0)
                gat(j1 + 1, rows0, g0)
                sca(j1, rows1, s1)
                sca_w(j1, rows1, s1)
                return carry2

            lax.fori_loop(0, cb // 2 - 1, pair, carry)
            j0 = cb - 2
            j1 = cb - 1
            gat_w(j0, rows0, g0)
            gat(j1, rows1, g1)
            sca(j0, rows0, s0)
            gat_w(j1, rows1, g1)
            sca_w(j0, rows0, s0)
            sca(j1, rows1, s1)
            sca_w(j1, rows1, s1)
            return carry

        lax.fori_loop(0, ch // cb, blk, 0)

        plsc.subcore_barrier()
        pltpu.sync_copy(acc.at[pl.ds(s * nt, nt)],
                        out_h.at[pl.ds(c * xoff + s * nt, nt)])

    return k


def _sc_count(nacc, w2, ch, xoff):
    mesh = plsc.VectorSubcoreMesh(core_axis_name="c", subcore_axis_name="s")
    nt = nacc // 16

    @functools.partial(
        pl.kernel,
        out_type=jax.ShapeDtypeStruct((2 * xoff, w2), jnp.float32),
        mesh=mesh,
        scratch_types=[
            pltpu.VMEM((ch, 128), jnp.int32),
            pltpu.VMEM((128, w2), jnp.float32),
            pltpu.VMEM_SHARED((nacc, w2), jnp.float32),
        ],
    )
    def k(dst_h, one_h, zer_h, out_h, didx, rows, acc):
        c = lax.axis_index("c")
        s = lax.axis_index("s")
        pltpu.sync_copy(zer_h, acc.at[pl.ds(s * nt, nt)])
        pltpu.sync_copy(dst_h.at[c, s], didx)
        pltpu.sync_copy(one_h, rows)
        plsc.subcore_barrier()

        def step(j, carry):
            pltpu.sync_copy(rows, acc.at[didx.at[j]], add=True)
            return carry

        lax.fori_loop(0, ch, step, 0)

        plsc.subcore_barrier()
        pltpu.sync_copy(acc.at[pl.ds(s * nt, nt)],
                        out_h.at[pl.ds(c * xoff + s * nt, nt)])

    return k


def _tck_entry(x, deg2):
    def body(x_ref, d0_ref, d1_ref, y_ref, dis_ref):
        d = lax.rsqrt(d0_ref[:, 0:1] + d1_ref[:, 0:1] + 1.0)
        y_ref[...] = x_ref[...] * d
        dis_ref[...] = jnp.broadcast_to(d, (_BN, 16))

    return pl.pallas_call(
        body,
        grid=(_NB,),
        in_specs=[
            pl.BlockSpec((_BN, 128), lambda i: (i, 0)),
            pl.BlockSpec((_BN, 128), lambda i: (i, 0)),
            pl.BlockSpec((_BN, 128), lambda i: (_NBA + i, 0)),
        ],
        out_specs=[
            pl.BlockSpec((_BN, 128), lambda i: (i, 0)),
            pl.BlockSpec((_BN, 16), lambda i: (i, 0)),
        ],
        out_shape=[
            jax.ShapeDtypeStruct((_NR, 128), jnp.float32),
            jax.ShapeDtypeStruct((_NR, 16), jnp.float32),
        ],
    )(x, deg2, deg2)


def _tck_layer1(a_esplit, y1, dis16, W, b):
    def body(a0, a1, y, dis, w_ref, b_ref, o_ref):
        d = dis[:, 0:1]
        z = (a0[...] + a1[...] + y[...]) * d
        h = jnp.dot(z, w_ref[...], preferred_element_type=jnp.float32)
        h = jnp.maximum(h + b_ref[0:1, :], 0.0) * d
        o_ref[...] = h

    return pl.pallas_call(
        body,
        grid=(_NB, 2),
        in_specs=[
            pl.BlockSpec((_BN, 128), lambda i, c: (i, 0)),
            pl.BlockSpec((_BN, 128), lambda i, c: (_NBA + i, 0)),
            pl.BlockSpec((_BN, 128), lambda i, c: (i, 0)),
            pl.BlockSpec((_BN, 16), lambda i, c: (i, 0)),
            pl.BlockSpec((_D, 128), lambda i, c: (0, c)),
            pl.BlockSpec((8, 128), lambda i, c: (0, c)),
        ],
        out_specs=pl.BlockSpec((_BN, 128), lambda i, c: (c * _NB + i, 0)),
        out_shape=jax.ShapeDtypeStruct((2 * _NR, 128), jnp.float32),
    )(a_esplit, a_esplit, y1, dis16, W, b)


def _tck_layer(a_pair, y_pair, dis16, W, b, relu, post, w2i):
    nya = (2 * _NR) // _BN // 2

    def body(a0, a1, y0, y1, dis, w_ref, b_ref, o_ref):
        d = dis[:, 0:1]
        z = jnp.concatenate([a0[...] + y0[...], a1[...] + y1[...]], axis=1) * d
        h = jnp.dot(z, w_ref[...], preferred_element_type=jnp.float32)
        h = h + b_ref[0:1, :]
        if relu:
            h = jnp.maximum(h, 0.0)
        if post:
            h = h * d
        o_ref[...] = h

    wi = 2 * w2i
    return pl.pallas_call(
        body,
        grid=(_NB, 2),
        in_specs=[
            pl.BlockSpec((_BN, w2i), lambda i, c: (i, 0)),
            pl.BlockSpec((_BN, w2i), lambda i, c: (_NBA + i, 0)),
            pl.BlockSpec((_BN, w2i), lambda i, c: (i, 0)),
            pl.BlockSpec((_BN, w2i), lambda i, c: (nya + i, 0)),
            pl.BlockSpec((_BN, 16), lambda i, c: (i, 0)),
            pl.BlockSpec((wi, 128), lambda i, c: (0, c)),
            pl.BlockSpec((8, 128), lambda i, c: (0, c)),
        ],
        out_specs=pl.BlockSpec((_BN, 128), lambda i, c: (c * _NB + i, 0)),
        out_shape=jax.ShapeDtypeStruct((2 * _NR, 128), jnp.float32),
    )(a_pair, a_pair, y_pair, y_pair, dis16, W, b)


def _tck_head(s_pair, cnt2, Wl_pad, bl_pad):
    def body(s0, s1, cnt, w_ref, b_ref, o_ref):
        c = jnp.maximum(cnt[:, 0:1], 1.0)
        s = jnp.concatenate([s0[...], s1[...]], axis=1) / c
        o_ref[...] = (jnp.dot(s, w_ref[...], preferred_element_type=jnp.float32)
                      + b_ref[0:1, :])

    return pl.pallas_call(
        body,
        grid=(1,),
        in_specs=[
            pl.BlockSpec((_G, 128), lambda i: (0, 0)),
            pl.BlockSpec((_G, 128), lambda i: (2, 0)),
            pl.BlockSpec((_G, 128), lambda i: (0, 0)),
            pl.BlockSpec((_H, 128), lambda i: (0, 0)),
            pl.BlockSpec((8, 128), lambda i: (0, 0)),
        ],
        out_specs=pl.BlockSpec((_G, 128), lambda i: (0, 0)),
        out_shape=jax.ShapeDtypeStruct((_G, 128), jnp.float32),
    )(s_pair, s_pair, cnt2, Wl_pad, bl_pad)


def kernel(x, edge_index, batch, W1, b1, W2, b2, W3, b3, Wl, bl):
    f32 = jnp.float32
    src = edge_index[0].astype(jnp.int32)
    dst = edge_index[1].astype(jnp.int32)
    pad = _EP - _E
    srcp = jnp.concatenate([src, jnp.zeros((pad,), jnp.int32)])
    dstp = jnp.concatenate([dst, jnp.full((pad,), _NR, jnp.int32)])
    src2 = jnp.stack([srcp, srcp + _NR]).reshape(2, 16, _CH_E, 128)
    dst_dup = jnp.broadcast_to(
        dstp.reshape(1, 16, _CH_E, 128), (2, 16, _CH_E, 128))
    src_split = srcp.reshape(2, 16, _CH_DEG, 128)
    dst_split = dstp.reshape(2, 16, _CH_DEG, 128)

    nodes = jnp.arange(_NR, dtype=jnp.int32)
    padp = _EP_P - _NR
    nodesp = jnp.concatenate([nodes, jnp.zeros((padp,), jnp.int32)])
    src2_p = jnp.stack([nodesp, nodesp + _NR]).reshape(2, 16, _CH_P, 128)
    batchp = jnp.concatenate(
        [batch.astype(jnp.int32), jnp.full((padp,), _G, jnp.int32)])
    dst_p = jnp.broadcast_to(
        batchp.reshape(1, 16, _CH_P, 128), (2, 16, _CH_P, 128))

    ones128 = jnp.ones((128, 128), f32)
    z_a = jnp.zeros((_ACC // 16, 128), f32)
    z_p = jnp.zeros((_GACC // 16, 128), f32)

    b1p = jnp.broadcast_to(b1.reshape(1, _H), (8, _H))
    b2p = jnp.broadcast_to(b2.reshape(1, _H), (8, _H))
    b3p = jnp.broadcast_to(b3.reshape(1, _H), (8, _H))
    Wl_pad = jnp.zeros((_H, 128), f32).at[:, :_C].set(Wl)
    bl_pad = jnp.broadcast_to(
        jnp.zeros((128,), f32).at[:_C].set(bl).reshape(1, 128), (8, 128))

    deg2 = _sc_count(_ACC, 128, _CH_DEG, _XO)(dst_split, ones128, z_a)
    cnt2 = _sc_count(_GACC, 128, _CH_P, _GACC)(dst_p, ones128, z_p)

    y1, dis16 = _tck_entry(x, deg2)
    a1 = _sc_agg(_ACC, 128, _CH_DEG, _XO)(src_split, dst_split, y1, z_a)
    y2 = _tck_layer1(a1, y1, dis16, W1, b1p)
    a2 = _sc_agg(_ACC, 128, _CH_E, _XO)(src2, dst_dup, y2, z_a)
    y3 = _tck_layer(a2, y2, dis16, W2, b2p, relu=True, post=True, w2i=128)
    a3 = _sc_agg(_ACC, 128, _CH_E, _XO)(src2, dst_dup, y3, z_a)
    h3 = _tck_layer(a3, y3, dis16, W3, b3p, relu=False, post=False, w2i=128)
    s2 = _sc_agg(_GACC, 128, _CH_P, _GACC)(src2_p, dst_p, h3, z_p)
    out = _tck_head(s2, cnt2, Wl_pad, bl_pad)
    return out[:, :_C]


# speedup vs baseline: 16.6882x; 2.2253x over previous
import functools

import jax
import jax.numpy as jnp
from jax import lax
from jax.experimental import pallas as pl
from jax.experimental.pallas import tpu as pltpu
from jax.experimental.pallas import tpu_sc as plsc

_NR = 10000
_E = 320000
_EP = 327680
_CH_E = 160
_CH_DEG = 80
_CB = 40
_ACC = 10112
_XO = 10400
_BN = 400
_NB = _NR // _BN
_NBA = _XO // _BN
_G = 128
_H = 256
_D = 128
_C = 10


def _sc_agg(nacc, w2, ch, xoff):
    mesh = plsc.VectorSubcoreMesh(core_axis_name="c", subcore_axis_name="s")
    nt = nacc // 16

    @functools.partial(
        pl.kernel,
        out_type=jax.ShapeDtypeStruct((2 * xoff, w2), jnp.float32),
        mesh=mesh,
        scratch_types=[
            pltpu.VMEM((min(ch, _CB), 128), jnp.int32),
            pltpu.VMEM((min(ch, _CB), 128), jnp.int32),
            pltpu.VMEM((128, w2), jnp.float32),
            pltpu.VMEM((128, w2), jnp.float32),
            pltpu.VMEM_SHARED((nacc, w2), jnp.float32),
            pltpu.SemaphoreType.DMA,
            pltpu.SemaphoreType.DMA,
            pltpu.SemaphoreType.DMA,
            pltpu.SemaphoreType.DMA,
        ],
    )
    def k(src_h, dst_h, tab_h, zer_h, out_h, sidx, didx, rows0, rows1,
          acc, g0, g1, s0, s1):
        c = lax.axis_index("c")
        s = lax.axis_index("s")
        cb = min(ch, _CB)
        pltpu.sync_copy(zer_h, acc.at[pl.ds(s * nt, nt)])
        plsc.subcore_barrier()

        def gat(j, rows, sem):
            pltpu.async_copy(tab_h.at[sidx.at[j]], rows, sem)

        def gat_w(j, rows, sem):
            pltpu.make_async_copy(tab_h.at[sidx.at[j]], rows, sem).wait()

        def sca(j, rows, sem):
            pltpu.async_copy(rows, acc.at[didx.at[j]], sem, add=True)

        def sca_w(j, rows, sem):
            pltpu.make_async_copy(rows, acc.at[didx.at[j]], sem).wait()

        def blk(b, carry):
            pltpu.sync_copy(dst_h.at[c, s, pl.ds(b * cb, cb)], didx)
            pltpu.sync_copy(src_h.at[c, s, pl.ds(b * cb, cb)], sidx)
            gat(0, rows0, g0)

            def pair(t, carry2):
                j0 = 2 * t
                j1 = j0 + 1
                gat_w(j0, rows0, g0)
                gat(j1, rows1, g1)
                sca(j0, rows0, s0)
                gat_w(j1, rows1, g1)
                sca_w(j0, rows0, s0)
                gat(j1 + 1, rows0, g0)
                sca(j1, rows1, s1)
                sca_w(j1, rows1, s1)
                return carry2

            lax.fori_loop(0, cb // 2 - 1, pair, carry)
            j0 = cb - 2
            j1 = cb - 1
            gat_w(j0, rows0, g0)
            gat(j1, rows1, g1)
            sca(j0, rows0, s0)
            gat_w(j1, rows1, g1)
            sca_w(j0, rows0, s0)
            sca(j1, rows1, s1)
            sca_w(j1, rows1, s1)
            return carry

        lax.fori_loop(0, ch // cb, blk, 0)

        plsc.subcore_barrier()
        pltpu.sync_copy(acc.at[pl.ds(s * nt, nt)],
                        out_h.at[pl.ds(c * xoff + s * nt, nt)])

    return k


def _sc_count(nacc, w2, ch, xoff):
    mesh = plsc.VectorSubcoreMesh(core_axis_name="c", subcore_axis_name="s")
    nt = nacc // 16

    @functools.partial(
        pl.kernel,
        out_type=jax.ShapeDtypeStruct((2 * xoff, w2), jnp.float32),
        mesh=mesh,
        scratch_types=[
            pltpu.VMEM((ch, 128), jnp.int32),
            pltpu.VMEM((128, w2), jnp.float32),
            pltpu.VMEM_SHARED((nacc, w2), jnp.float32),
        ],
    )
    def k(dst_h, one_h, zer_h, out_h, didx, rows, acc):
        c = lax.axis_index("c")
        s = lax.axis_index("s")
        pltpu.sync_copy(zer_h, acc.at[pl.ds(s * nt, nt)])
        pltpu.sync_copy(dst_h.at[c, s], didx)
        pltpu.sync_copy(one_h, rows)
        plsc.subcore_barrier()

        def step(j, carry):
            pltpu.sync_copy(rows, acc.at[didx.at[j]], add=True)
            return carry

        lax.fori_loop(0, ch, step, 0)

        plsc.subcore_barrier()
        pltpu.sync_copy(acc.at[pl.ds(s * nt, nt)],
                        out_h.at[pl.ds(c * xoff + s * nt, nt)])

    return k


def _tck_entry(x, deg2):
    def body(x_ref, d0_ref, d1_ref, y_ref, dis_ref):
        d = lax.rsqrt(d0_ref[:, 0:1] + d1_ref[:, 0:1] + 1.0)
        y_ref[...] = x_ref[...] * d
        dis_ref[...] = jnp.broadcast_to(d, (_BN, 16))

    return pl.pallas_call(
        body,
        grid=(_NB,),
        in_specs=[
            pl.BlockSpec((_BN, 128), lambda i: (i, 0)),
            pl.BlockSpec((_BN, 128), lambda i: (i, 0)),
            pl.BlockSpec((_BN, 128), lambda i: (_NBA + i, 0)),
        ],
        out_specs=[
            pl.BlockSpec((_BN, 128), lambda i: (i, 0)),
            pl.BlockSpec((_BN, 16), lambda i: (i, 0)),
        ],
        out_shape=[
            jax.ShapeDtypeStruct((_NR, 128), jnp.float32),
            jax.ShapeDtypeStruct((_NR, 16), jnp.float32),
        ],
    )(x, deg2, deg2)


def _tck_layer1(a_esplit, y1, dis16, W, b):
    def body(a0, a1, y, dis, w_ref, b_ref, o_ref):
        d = dis[:, 0:1]
        z = (a0[...] + a1[...] + y[...]) * d
        h = jnp.dot(z, w_ref[...], preferred_element_type=jnp.float32)
        h = jnp.maximum(h + b_ref[0:1, :], 0.0) * d
        o_ref[...] = h

    return pl.pallas_call(
        body,
        grid=(_NB, 2),
        in_specs=[
            pl.BlockSpec((_BN, 128), lambda i, c: (i, 0)),
            pl.BlockSpec((_BN, 128), lambda i, c: (_NBA + i, 0)),
            pl.BlockSpec((_BN, 128), lambda i, c: (i, 0)),
            pl.BlockSpec((_BN, 16), lambda i, c: (i, 0)),
            pl.BlockSpec((_D, 128), lambda i, c: (0, c)),
            pl.BlockSpec((8, 128), lambda i, c: (0, c)),
        ],
        out_specs=pl.BlockSpec((_BN, 128), lambda i, c: (c * _NB + i, 0)),
        out_shape=jax.ShapeDtypeStruct((2 * _NR, 128), jnp.float32),
    )(a_esplit, a_esplit, y1, dis16, W, b)


def _tck_layer(a_pair, y_pair, dis16, W, b, relu, post, w2i):
    nya = (2 * _NR) // _BN // 2

    def body(a0, a1, y0, y1, dis, w_ref, b_ref, o_ref):
        d = dis[:, 0:1]
        z = jnp.concatenate([a0[...] + y0[...], a1[...] + y1[...]], axis=1) * d
        h = jnp.dot(z, w_ref[...], preferred_element_type=jnp.float32)
        h = h + b_ref[0:1, :]
        if relu:
            h = jnp.maximum(h, 0.0)
        if post:
            h = h * d
        o_ref[...] = h

    wi = 2 * w2i
    return pl.pallas_call(
        body,
        grid=(_NB, 2),
        in_specs=[
            pl.BlockSpec((_BN, w2i), lambda i, c: (i, 0)),
            pl.BlockSpec((_BN, w2i), lambda i, c: (_NBA + i, 0)),
            pl.BlockSpec((_BN, w2i), lambda i, c: (i, 0)),
            pl.BlockSpec((_BN, w2i), lambda i, c: (nya + i, 0)),
            pl.BlockSpec((_BN, 16), lambda i, c: (i, 0)),
            pl.BlockSpec((wi, 128), lambda i, c: (0, c)),
            pl.BlockSpec((8, 128), lambda i, c: (0, c)),
        ],
        out_specs=pl.BlockSpec((_BN, 128), lambda i, c: (c * _NB + i, 0)),
        out_shape=jax.ShapeDtypeStruct((2 * _NR, 128), jnp.float32),
    )(a_pair, a_pair, y_pair, y_pair, dis16, W, b)


def _tck_pool(h_pair, batch16):
    def body(h0, h1, bat, s_ref, c_ref):
        i = pl.program_id(0)
        seg = bat[:, 0:1]
        gid = lax.broadcasted_iota(jnp.int32, (_BN, _G), 1)
        S = (seg == gid).astype(jnp.float32)
        p0 = lax.dot_general(S, h0[...], (((0,), (0,)), ((), ())),
                             preferred_element_type=jnp.float32)
        p1 = lax.dot_general(S, h1[...], (((0,), (0,)), ((), ())),
                             preferred_element_type=jnp.float32)
        p = jnp.concatenate([p0, p1], axis=1)
        cnt = lax.dot_general(S, jnp.ones((_BN, 16), jnp.float32),
                              (((0,), (0,)), ((), ())),
                              preferred_element_type=jnp.float32)

        @pl.when(i == 0)
        def _():
            s_ref[...] = p
            c_ref[...] = cnt

        @pl.when(i > 0)
        def _():
            s_ref[...] += p
            c_ref[...] += cnt

    return pl.pallas_call(
        body,
        grid=(_NB,),
        in_specs=[
            pl.BlockSpec((_BN, 128), lambda i: (i, 0)),
            pl.BlockSpec((_BN, 128), lambda i: (_NB + i, 0)),
            pl.BlockSpec((_BN, 16), lambda i: (i, 0)),
        ],
        out_specs=[
            pl.BlockSpec((_G, 256), lambda i: (0, 0)),
            pl.BlockSpec((_G, 16), lambda i: (0, 0)),
        ],
        out_shape=[
            jax.ShapeDtypeStruct((_G, 256), jnp.float32),
            jax.ShapeDtypeStruct((_G, 16), jnp.float32),
        ],
    )(h_pair, h_pair, batch16)


def _tck_head(sums, cnt, Wl_pad, bl_pad):
    def body(s_ref, c_ref, w_ref, b_ref, o_ref):
        c = jnp.maximum(c_ref[:, 0:1], 1.0)
        s = s_ref[...] / c
        o_ref[...] = (jnp.dot(s, w_ref[...], preferred_element_type=jnp.float32)
                      + b_ref[0:1, :])

    return pl.pallas_call(
        body,
        grid=(1,),
        in_specs=[
            pl.BlockSpec((_G, 256), lambda i: (0, 0)),
            pl.BlockSpec((_G, 16), lambda i: (0, 0)),
            pl.BlockSpec((_H, 128), lambda i: (0, 0)),
            pl.BlockSpec((8, 128), lambda i: (0, 0)),
        ],
        out_specs=pl.BlockSpec((_G, 128), lambda i: (0, 0)),
        out_shape=jax.ShapeDtypeStruct((_G, 128), jnp.float32),
    )(sums, cnt, Wl_pad, bl_pad)


def kernel(x, edge_index, batch, W1, b1, W2, b2, W3, b3, Wl, bl):
    f32 = jnp.float32
    src = edge_index[0].astype(jnp.int32)
    dst = edge_index[1].astype(jnp.int32)
    pad = _EP - _E
    padi = jnp.arange(pad, dtype=jnp.int32)
    srcp = jnp.concatenate([src, padi * 37 % _NR])
    dstp = jnp.concatenate([dst, _NR + padi % (_ACC - _NR)])
    src2 = jnp.stack([srcp, srcp + _NR]).reshape(2, 16, _CH_E, 128)
    dst_dup = jnp.broadcast_to(
        dstp.reshape(1, 16, _CH_E, 128), (2, 16, _CH_E, 128))
    src_split = srcp.reshape(2, 16, _CH_DEG, 128)
    dst_split = dstp.reshape(2, 16, _CH_DEG, 128)

    ones128 = jnp.ones((128, 128), f32)
    z_a = jnp.zeros((_ACC // 16, 128), f32)

    batch16 = jnp.broadcast_to(
        batch.astype(jnp.int32).reshape(_NR, 1), (_NR, 16))

    b1p = jnp.broadcast_to(b1.reshape(1, _H), (8, _H))
    b2p = jnp.broadcast_to(b2.reshape(1, _H), (8, _H))
    b3p = jnp.broadcast_to(b3.reshape(1, _H), (8, _H))
    Wl_pad = jnp.zeros((_H, 128), f32).at[:, :_C].set(Wl)
    bl_pad = jnp.broadcast_to(
        jnp.zeros((128,), f32).at[:_C].set(bl).reshape(1, 128), (8, 128))

    deg2 = _sc_count(_ACC, 128, _CH_DEG, _XO)(dst_split, ones128, z_a)

    y1, dis16 = _tck_entry(x, deg2)
    a1 = _sc_agg(_ACC, 128, _CH_DEG, _XO)(src_split, dst_split, y1, z_a)
    y2 = _tck_layer1(a1, y1, dis16, W1, b1p)
    a2 = _sc_agg(_ACC, 128, _CH_E, _XO)(src2, dst_dup, y2, z_a)
    y3 = _tck_layer(a2, y2, dis16, W2, b2p, relu=True, post=True, w2i=128)
    a3 = _sc_agg(_ACC, 128, _CH_E, _XO)(src2, dst_dup, y3, z_a)
    h3 = _tck_layer(a3, y3, dis16, W3, b3p, relu=False, post=False, w2i=128)
    sums, cnt = _tck_pool(h3, batch16)
    out = _tck_head(sums, cnt, Wl_pad, bl_pad)
    return out[:, :_C]


# speedup vs baseline: 17.5540x; 1.0519x over previous
import functools

import jax
import jax.numpy as jnp
from jax import lax
from jax.experimental import pallas as pl
from jax.experimental.pallas import tpu as pltpu
from jax.experimental.pallas import tpu_sc as plsc

_NR = 10000
_E = 320000
_EP = 327680
_CH_E = 160
_CH_DEG = 80
_CB = 40
_ACC = 10112
_XO = 10400
_BN = 400
_NB = _NR // _BN
_NBA = _XO // _BN
_G = 128
_H = 256
_D = 128
_C = 10


def _sc_agg(nacc, w2, ch, xoff):
    mesh = plsc.VectorSubcoreMesh(core_axis_name="c", subcore_axis_name="s")
    nt = nacc // 16

    @functools.partial(
        pl.kernel,
        out_type=jax.ShapeDtypeStruct((2 * xoff, w2), jnp.float32),
        mesh=mesh,
        scratch_types=[
            pltpu.VMEM((min(ch, _CB), 128), jnp.int32),
            pltpu.VMEM((min(ch, _CB), 128), jnp.int32),
            pltpu.VMEM((128, w2), jnp.float32),
            pltpu.VMEM((128, w2), jnp.float32),
            pltpu.VMEM_SHARED((nacc, w2), jnp.float32),
            pltpu.SemaphoreType.DMA,
            pltpu.SemaphoreType.DMA,
            pltpu.SemaphoreType.DMA,
            pltpu.SemaphoreType.DMA,
        ],
    )
    def k(src_h, dst_h, tab_h, zer_h, out_h, sidx, didx, rows0, rows1,
          acc, g0, g1, s0, s1):
        c = lax.axis_index("c")
        s = lax.axis_index("s")
        cb = min(ch, _CB)
        pltpu.sync_copy(zer_h, acc.at[pl.ds(s * nt, nt)])
        plsc.subcore_barrier()

        def gat(j, rows, sem):
            pltpu.async_copy(tab_h.at[sidx.at[j]], rows, sem)

        def gat_w(j, rows, sem):
            pltpu.make_async_copy(tab_h.at[sidx.at[j]], rows, sem).wait()

        def sca(j, rows, sem):
            pltpu.async_copy(rows, acc.at[didx.at[j]], sem, add=True)

        def sca_w(j, rows, sem):
            pltpu.make_async_copy(rows, acc.at[didx.at[j]], sem).wait()

        def blk(b, carry):
            pltpu.sync_copy(dst_h.at[c, s, pl.ds(b * cb, cb)], didx)
            pltpu.sync_copy(src_h.at[c, s, pl.ds(b * cb, cb)], sidx)
            gat(0, rows0, g0)

            def pair(t, carry2):
                j0 = 2 * t
                j1 = j0 + 1
                gat_w(j0, rows0, g0)
                gat(j1, rows1, g1)
                sca(j0, rows0, s0)
                gat_w(j1, rows1, g1)
                sca_w(j0, rows0, s0)
                gat(j1 + 1, rows0, g0)
                sca(j1, rows1, s1)
                sca_w(j1, rows1, s1)
                return carry2

            lax.fori_loop(0, cb // 2 - 1, pair, carry)
            j0 = cb - 2
            j1 = cb - 1
            gat_w(j0, rows0, g0)
            gat(j1, rows1, g1)
            sca(j0, rows0, s0)
            gat_w(j1, rows1, g1)
            sca_w(j0, rows0, s0)
            sca(j1, rows1, s1)
            sca_w(j1, rows1, s1)
            return carry

        lax.fori_loop(0, ch // cb, blk, 0)

        plsc.subcore_barrier()
        pltpu.sync_copy(acc.at[pl.ds(s * nt, nt)],
                        out_h.at[pl.ds(c * xoff + s * nt, nt)])

    return k


def _sc_count(nacc, w2, ch, xoff):
    mesh = plsc.VectorSubcoreMesh(core_axis_name="c", subcore_axis_name="s")
    nt = nacc // 16

    @functools.partial(
        pl.kernel,
        out_type=jax.ShapeDtypeStruct((2 * xoff, w2), jnp.float32),
        mesh=mesh,
        scratch_types=[
            pltpu.VMEM((ch, 128), jnp.int32),
            pltpu.VMEM((128, w2), jnp.float32),
            pltpu.VMEM_SHARED((nacc, w2), jnp.float32),
        ],
    )
    def k(dst_h, one_h, zer_h, out_h, didx, rows, acc):
        c = lax.axis_index("c")
        s = lax.axis_index("s")
        pltpu.sync_copy(zer_h, acc.at[pl.ds(s * nt, nt)])
        pltpu.sync_copy(dst_h.at[c, s], didx)
        pltpu.sync_copy(one_h, rows)
        plsc.subcore_barrier()

        def step(j, carry):
            pltpu.sync_copy(rows, acc.at[didx.at[j]], add=True)
            return carry

        lax.fori_loop(0, ch, step, 0)

        plsc.subcore_barrier()
        pltpu.sync_copy(acc.at[pl.ds(s * nt, nt)],
                        out_h.at[pl.ds(c * xoff + s * nt, nt)])

    return k


def _tck_entry(x, deg2):
    def body(x_ref, d0_ref, d1_ref, y_ref, dis_ref):
        d = lax.rsqrt(d0_ref[:, 0:1] + d1_ref[:, 0:1] + 1.0)
        y_ref[...] = x_ref[...] * d
        dis_ref[...] = jnp.broadcast_to(d, (_BN, 16))

    return pl.pallas_call(
        body,
        grid=(_NB,),
        in_specs=[
            pl.BlockSpec((_BN, 128), lambda i: (i, 0)),
            pl.BlockSpec((_BN, 128), lambda i: (i, 0)),
            pl.BlockSpec((_BN, 128), lambda i: (_NBA + i, 0)),
        ],
        out_specs=[
            pl.BlockSpec((_BN, 128), lambda i: (i, 0)),
            pl.BlockSpec((_BN, 16), lambda i: (i, 0)),
        ],
        out_shape=[
            jax.ShapeDtypeStruct((_NR, 128), jnp.float32),
            jax.ShapeDtypeStruct((_NR, 16), jnp.float32),
        ],
    )(x, deg2, deg2)


def _tck_layer1(a_esplit, y1, dis16, W, b):
    def body(a0, a1, y, dis, w_ref, b_ref, o_ref):
        d = dis[:, 0:1]
        z = (a0[...] + a1[...] + y[...]) * d
        h = jnp.dot(z, w_ref[...], preferred_element_type=jnp.float32)
        h = jnp.maximum(h + b_ref[0:1, :], 0.0) * d
        o_ref[...] = h

    return pl.pallas_call(
        body,
        grid=(_NB, 2),
        in_specs=[
            pl.BlockSpec((_BN, 128), lambda i, c: (i, 0)),
            pl.BlockSpec((_BN, 128), lambda i, c: (_NBA + i, 0)),
            pl.BlockSpec((_BN, 128), lambda i, c: (i, 0)),
            pl.BlockSpec((_BN, 16), lambda i, c: (i, 0)),
            pl.BlockSpec((_D, 128), lambda i, c: (0, c)),
            pl.BlockSpec((8, 128), lambda i, c: (0, c)),
        ],
        out_specs=pl.BlockSpec((_BN, 128), lambda i, c: (c * _NB + i, 0)),
        out_shape=jax.ShapeDtypeStruct((2 * _NR, 128), jnp.float32),
    )(a_esplit, a_esplit, y1, dis16, W, b)


def _tck_layer(a_pair, y_pair, dis16, W, b, relu, post, w2i):
    nya = (2 * _NR) // _BN // 2

    def body(a0, a1, y0, y1, dis, w_ref, b_ref, o_ref):
        d = dis[:, 0:1]
        z = jnp.concatenate([a0[...] + y0[...], a1[...] + y1[...]], axis=1) * d
        h = jnp.dot(z, w_ref[...], preferred_element_type=jnp.float32)
        h = h + b_ref[0:1, :]
        if relu:
            h = jnp.maximum(h, 0.0)
        if post:
            h = h * d
        o_ref[...] = h

    wi = 2 * w2i
    return pl.pallas_call(
        body,
        grid=(_NB, 2),
        in_specs=[
            pl.BlockSpec((_BN, w2i), lambda i, c: (i, 0)),
            pl.BlockSpec((_BN, w2i), lambda i, c: (_NBA + i, 0)),
            pl.BlockSpec((_BN, w2i), lambda i, c: (i, 0)),
            pl.BlockSpec((_BN, w2i), lambda i, c: (nya + i, 0)),
            pl.BlockSpec((_BN, 16), lambda i, c: (i, 0)),
            pl.BlockSpec((wi, 128), lambda i, c: (0, c)),
            pl.BlockSpec((8, 128), lambda i, c: (0, c)),
        ],
        out_specs=pl.BlockSpec((_BN, 128), lambda i, c: (c * _NB + i, 0)),
        out_shape=jax.ShapeDtypeStruct((2 * _NR, 128), jnp.float32),
    )(a_pair, a_pair, y_pair, y_pair, dis16, W, b)


def _tck_l3_pool_head(a_pair, y_pair, dis16, batch16, W, b, Wl_pad, bl_pad):
    nya = (2 * _NR) // _BN // 2

    def body(a0, a1, y0, y1, dis, bat, w_ref, b_ref, wl_ref, bl_ref,
             o_ref, s_ref, c_ref):
        i = pl.program_id(0)
        d = dis[:, 0:1]
        z = jnp.concatenate([a0[...] + y0[...], a1[...] + y1[...]], axis=1) * d
        h = jnp.dot(z, w_ref[...], preferred_element_type=jnp.float32)
        h = h + b_ref[0:1, :]
        seg = bat[:, 0:1]
        gid = lax.broadcasted_iota(jnp.int32, (_BN, _G), 1)
        S = (seg == gid).astype(jnp.float32)
        p = lax.dot_general(S, h, (((0,), (0,)), ((), ())),
                            preferred_element_type=jnp.float32)
        cnt = lax.dot_general(S, jnp.ones((_BN, 16), jnp.float32),
                              (((0,), (0,)), ((), ())),
                              preferred_element_type=jnp.float32)

        @pl.when(i == 0)
        def _():
            s_ref[...] = p
            c_ref[...] = cnt

        @pl.when(i > 0)
        def _():
            s_ref[...] += p
            c_ref[...] += cnt

        @pl.when(i == _NB - 1)
        def _():
            cc = jnp.maximum(c_ref[:, 0:1], 1.0)
            sfin = s_ref[...] / cc
            o_ref[...] = (jnp.dot(sfin, wl_ref[...],
                                  preferred_element_type=jnp.float32)
                          + bl_ref[0:1, :])

    out, _, _ = pl.pallas_call(
        body,
        grid=(_NB,),
        in_specs=[
            pl.BlockSpec((_BN, 128), lambda i: (i, 0)),
            pl.BlockSpec((_BN, 128), lambda i: (_NBA + i, 0)),
            pl.BlockSpec((_BN, 128), lambda i: (i, 0)),
            pl.BlockSpec((_BN, 128), lambda i: (nya + i, 0)),
            pl.BlockSpec((_BN, 16), lambda i: (i, 0)),
            pl.BlockSpec((_BN, 16), lambda i: (i, 0)),
            pl.BlockSpec((_H, 256), lambda i: (0, 0)),
            pl.BlockSpec((8, 256), lambda i: (0, 0)),
            pl.BlockSpec((_H, 128), lambda i: (0, 0)),
            pl.BlockSpec((8, 128), lambda i: (0, 0)),
        ],
        out_specs=[
            pl.BlockSpec((_G, 128), lambda i: (0, 0)),
            pl.BlockSpec((_G, 256), lambda i: (0, 0)),
            pl.BlockSpec((_G, 16), lambda i: (0, 0)),
        ],
        out_shape=[
            jax.ShapeDtypeStruct((_G, 128), jnp.float32),
            jax.ShapeDtypeStruct((_G, 256), jnp.float32),
            jax.ShapeDtypeStruct((_G, 16), jnp.float32),
        ],
    )(a_pair, a_pair, y_pair, y_pair, dis16, batch16, W, b, Wl_pad, bl_pad)
    return out


def kernel(x, edge_index, batch, W1, b1, W2, b2, W3, b3, Wl, bl):
    f32 = jnp.float32
    src = edge_index[0].astype(jnp.int32)
    dst = edge_index[1].astype(jnp.int32)
    pad = _EP - _E
    padi = jnp.arange(pad, dtype=jnp.int32)
    srcp = jnp.concatenate([src, padi * 37 % _NR])
    dstp = jnp.concatenate([dst, _NR + padi % (_ACC - _NR)])
    src2 = jnp.stack([srcp, srcp + _NR]).reshape(2, 16, _CH_E, 128)
    dst_dup = jnp.broadcast_to(
        dstp.reshape(1, 16, _CH_E, 128), (2, 16, _CH_E, 128))
    src_split = srcp.reshape(2, 16, _CH_DEG, 128)
    dst_split = dstp.reshape(2, 16, _CH_DEG, 128)

    ones128 = jnp.ones((128, 128), f32)
    z_a = jnp.zeros((_ACC // 16, 128), f32)

    batch16 = jnp.broadcast_to(
        batch.astype(jnp.int32).reshape(_NR, 1), (_NR, 16))

    b1p = jnp.broadcast_to(b1.reshape(1, _H), (8, _H))
    b2p = jnp.broadcast_to(b2.reshape(1, _H), (8, _H))
    b3p = jnp.broadcast_to(b3.reshape(1, _H), (8, _H))
    Wl_pad = jnp.zeros((_H, 128), f32).at[:, :_C].set(Wl)
    bl_pad = jnp.broadcast_to(
        jnp.zeros((128,), f32).at[:_C].set(bl).reshape(1, 128), (8, 128))

    deg2 = _sc_count(_ACC, 128, _CH_DEG, _XO)(dst_split, ones128, z_a)

    y1, dis16 = _tck_entry(x, deg2)
    a1 = _sc_agg(_ACC, 128, _CH_DEG, _XO)(src_split, dst_split, y1, z_a)
    y2 = _tck_layer1(a1, y1, dis16, W1, b1p)
    a2 = _sc_agg(_ACC, 128, _CH_E, _XO)(src2, dst_dup, y2, z_a)
    y3 = _tck_layer(a2, y2, dis16, W2, b2p, relu=True, post=True, w2i=128)
    a3 = _sc_agg(_ACC, 128, _CH_E, _XO)(src2, dst_dup, y3, z_a)
    out = _tck_l3_pool_head(a3, y3, dis16, batch16, W3, b3p, Wl_pad, bl_pad)
    return out[:, :_C]
